# baseline (speedup 1.0000x reference)
import functools

import jax
import jax.numpy as jnp
from jax.experimental import pallas as pl
from jax.experimental.pallas import tpu as pltpu

APP = 128
STW = 4 + 17 * 3
FEAT = APP + 1 + STW
TOK = 64
ROWS_PER_BLOCK = 2048


def _proj_body(emb_ref, vis_ref, bbox_ref, kpts_ref, mask_ref, w_ref, b_ref, out_ref):
    w = w_ref[...]
    acc = jnp.dot(emb_ref[...], w[0:APP, :], preferred_element_type=jnp.float32)
    acc += vis_ref[...] * w[APP:APP + 1, :]
    acc += jnp.dot(bbox_ref[...], w[APP + 1:APP + 5, :],
                   preferred_element_type=jnp.float32)
    acc += jnp.dot(kpts_ref[...], w[APP + 5:FEAT, :],
                   preferred_element_type=jnp.float32)
    acc += b_ref[...]
    out_ref[...] = jnp.where(mask_ref[...] != 0, acc, 0.0)


@functools.partial(jax.jit, static_argnames=())
def kernel(embeddings, visibility_scores, bbox_ltwh, keypoints_xyc, feats_masks, W, b):
    Bm, Nm = feats_masks.shape
    M = Bm * Nm
    R = ROWS_PER_BLOCK
    emb2 = embeddings.reshape(M, APP)
    vis2 = visibility_scores.reshape(M, 1)
    bbox2 = bbox_ltwh.reshape(M, 4)
    kpts2 = keypoints_xyc.reshape(M, STW - 4)
    mask2 = feats_masks.reshape(M, 1).astype(jnp.float32)
    b2 = b.reshape(1, TOK)

    grid = (M // R,)
    out = pl.pallas_call(
        _proj_body,
        grid=grid,
        in_specs=[
            pl.BlockSpec((R, APP), lambda i: (i, 0)),
            pl.BlockSpec((R, 1), lambda i: (i, 0)),
            pl.BlockSpec((R, 4), lambda i: (i, 0)),
            pl.BlockSpec((R, STW - 4), lambda i: (i, 0)),
            pl.BlockSpec((R, 1), lambda i: (i, 0)),
            pl.BlockSpec((FEAT, TOK), lambda i: (0, 0)),
            pl.BlockSpec((1, TOK), lambda i: (0, 0)),
        ],
        out_specs=pl.BlockSpec((R, TOK), lambda i: (i, 0)),
        out_shape=jax.ShapeDtypeStruct((M, TOK), jnp.float32),
        compiler_params=pltpu.CompilerParams(
            dimension_semantics=("arbitrary",),
        ),
    )(emb2, vis2, bbox2, kpts2, mask2, W, b2)
    return out.reshape(Bm, Nm, TOK)


# speedup vs baseline: 2.8357x; 2.8357x over previous
import jax
import jax.numpy as jnp
from jax.experimental import pallas as pl
from jax.experimental.pallas import tpu as pltpu

APP = 128
NKPT = 51
NST = 56
FEAT = APP + NST
TOK = 64
BSUB = 8
RN = 512


def _proj_body(emb_ref, vis_ref, bbox_ref, kpts_ref, mask_ref, w_ref, wst_ref,
               bias_ref, out_ref):
    w_e = w_ref[0:APP, :]
    emb2 = emb_ref[...].reshape(BSUB * RN, APP)
    acc = jnp.dot(emb2, w_e, preferred_element_type=jnp.float32)
    acc = (acc + bias_ref[...]).reshape(BSUB, RN, TOK)
    for i in range(BSUB):
        st_t = jnp.concatenate(
            [vis_ref[i], bbox_ref[i], kpts_ref[:, i, :]], axis=0)
        st_o_t = jnp.dot(wst_ref[...], st_t,
                         preferred_element_type=jnp.float32)
        z = jnp.concatenate([st_o_t, mask_ref[i:i + 1, :]], axis=0)
        zt = jnp.transpose(z, (1, 0))
        out_ref[i] = jnp.where(zt[:, TOK:TOK + 1] != 0,
                               acc[i] + zt[:, 0:TOK], 0.0)


def kernel(embeddings, visibility_scores, bbox_ltwh, keypoints_xyc, feats_masks, W, b):
    Bm, Nm = feats_masks.shape
    vis_t = jnp.transpose(visibility_scores, (0, 2, 1))
    bbox_t = jnp.transpose(bbox_ltwh, (0, 2, 1))
    kpts_t = jnp.transpose(keypoints_xyc, (2, 3, 0, 1)).reshape(NKPT, Bm, Nm)
    mask_f = feats_masks.astype(jnp.float32)
    wst_t = jnp.transpose(W[APP:FEAT, :], (1, 0))
    b2 = b.reshape(1, TOK)

    grid = (Bm // BSUB, Nm // RN)
    out = pl.pallas_call(
        _proj_body,
        grid=grid,
        in_specs=[
            pl.BlockSpec((BSUB, RN, APP), lambda i, j: (i, j, 0)),
            pl.BlockSpec((BSUB, 1, RN), lambda i, j: (i, 0, j)),
            pl.BlockSpec((BSUB, 4, RN), lambda i, j: (i, 0, j)),
            pl.BlockSpec((NKPT, BSUB, RN), lambda i, j: (0, i, j)),
            pl.BlockSpec((BSUB, RN), lambda i, j: (i, j)),
            pl.BlockSpec((FEAT, TOK), lambda i, j: (0, 0)),
            pl.BlockSpec((TOK, NST), lambda i, j: (0, 0)),
            pl.BlockSpec((1, TOK), lambda i, j: (0, 0)),
        ],
        out_specs=pl.BlockSpec((BSUB, RN, TOK), lambda i, j: (i, j, 0)),
        out_shape=jax.ShapeDtypeStruct((Bm, Nm, TOK), jnp.float32),
        compiler_params=pltpu.CompilerParams(
            dimension_semantics=("parallel", "arbitrary"),
        ),
    )(embeddings, vis_t, bbox_t, kpts_t, mask_f, W, wst_t, b2)
    return out


# speedup vs baseline: 3.0216x; 1.0655x over previous
import jax
import jax.numpy as jnp
from jax.experimental import pallas as pl
from jax.experimental.pallas import tpu as pltpu

APP = 128
NKPT = 51
NST = 56
FEAT = APP + NST
TOK = 64
BSUB = 8
RN = 1024


def _proj_body(emb_ref, vis_ref, bbox_ref, kpts_ref, mask_ref, w_ref, wst_ref,
               bias_ref, out_ref):
    w_e = w_ref[0:APP, :]
    emb2 = emb_ref[...].reshape(BSUB * RN, APP)
    acc = jnp.dot(emb2, w_e, preferred_element_type=jnp.float32)
    acc = (acc + bias_ref[...]).reshape(BSUB, RN, TOK)
    for i in range(BSUB):
        st_t = jnp.concatenate(
            [vis_ref[i], bbox_ref[i], kpts_ref[:, i, :]], axis=0)
        st_o_t = jnp.dot(wst_ref[...], st_t,
                         preferred_element_type=jnp.float32)
        z = jnp.concatenate([st_o_t, mask_ref[i:i + 1, :]], axis=0)
        zt = jnp.transpose(z, (1, 0))
        out_ref[i] = jnp.where(zt[:, TOK:TOK + 1] != 0,
                               acc[i] + zt[:, 0:TOK], 0.0)


def kernel(embeddings, visibility_scores, bbox_ltwh, keypoints_xyc, feats_masks, W, b):
    Bm, Nm = feats_masks.shape
    vis_t = jnp.transpose(visibility_scores, (0, 2, 1))
    bbox_t = jnp.transpose(bbox_ltwh, (0, 2, 1))
    kpts_t = jnp.transpose(keypoints_xyc, (2, 3, 0, 1)).reshape(NKPT, Bm, Nm)
    mask_f = feats_masks.astype(jnp.float32)
    wst_t = jnp.transpose(W[APP:FEAT, :], (1, 0))
    b2 = b.reshape(1, TOK)

    grid = (Bm // BSUB, Nm // RN)
    out = pl.pallas_call(
        _proj_body,
        grid=grid,
        in_specs=[
            pl.BlockSpec((BSUB, RN, APP), lambda i, j: (i, j, 0)),
            pl.BlockSpec((BSUB, 1, RN), lambda i, j: (i, 0, j)),
            pl.BlockSpec((BSUB, 4, RN), lambda i, j: (i, 0, j)),
            pl.BlockSpec((NKPT, BSUB, RN), lambda i, j: (0, i, j)),
            pl.BlockSpec((BSUB, RN), lambda i, j: (i, j)),
            pl.BlockSpec((FEAT, TOK), lambda i, j: (0, 0)),
            pl.BlockSpec((TOK, NST), lambda i, j: (0, 0)),
            pl.BlockSpec((1, TOK), lambda i, j: (0, 0)),
        ],
        out_specs=pl.BlockSpec((BSUB, RN, TOK), lambda i, j: (i, j, 0)),
        out_shape=jax.ShapeDtypeStruct((Bm, Nm, TOK), jnp.float32),
        compiler_params=pltpu.CompilerParams(
            dimension_semantics=("parallel", "arbitrary"),
        ),
    )(embeddings, vis_t, bbox_t, kpts_t, mask_f, W, wst_t, b2)
    return out
